# baseline (device time: 436385 ns/iter reference)
import jax
import jax.numpy as jnp
from jax import lax
from jax.experimental import pallas as pl
from jax.experimental.pallas import tpu as pltpu

N_ROWS = 4096
N_COLS = 4096
Q_ROWS = N_ROWS // 4
SIZES = [32, 64, 128, 160, 160, 160, 160, 160]
OFFS = [sum(SIZES[:i]) for i in range(len(SIZES))]
NC = len(SIZES)
LAST = NC - 1
REROUTE = 48
MAXCH = max(SIZES)

assert sum(SIZES) == Q_ROWS and REROUTE < SIZES[LAST]


def kernel(partial, resid, gamma):
    g = gamma.reshape(1, N_COLS)

    def body(p_ref, resid_ref, g_ref, out_ref,
             p_vmem, s_vmem, resid_vmem, o_vmem,
             p1_send, p1_recv, a_send, a_recv,
             b_own_send, b_own_recv, b_fwd_send, b_fwd_recv,
             c_send, c_recv, copy_sems):
        my_x = lax.axis_index("x")
        my_y = lax.axis_index("y")
        y_nbr = (my_x, 1 - my_y)
        x_nbr = (1 - my_x, my_y)

        q_me = 2 * my_x + my_y
        q_yn = 2 * my_x + (1 - my_y)
        q_xn = 2 * (1 - my_x) + my_y
        q_dg = 2 * (1 - my_x) + (1 - my_y)
        r_me = q_me * Q_ROWS
        r_yn = q_yn * Q_ROWS

        def rdma(src, dst, ssem, rsem, dev):
            return pltpu.make_async_remote_copy(
                src_ref=src, dst_ref=dst, send_sem=ssem, recv_sem=rsem,
                device_id=dev, device_id_type=pl.DeviceIdType.MESH)

        barrier = pltpu.get_barrier_semaphore()
        for nbr in (y_nbr, x_nbr):
            pl.semaphore_signal(barrier, inc=1, device_id=nbr,
                                device_id_type=pl.DeviceIdType.MESH)
        pl.semaphore_wait(barrier, 2)

        for c in range(NC):
            rows = pl.ds(r_yn + OFFS[c], SIZES[c])
            rdma(p_ref.at[0, rows], out_ref.at[rows],
                 p1_send.at[c], p1_recv.at[c], y_nbr).start()

        for c in range(NC):
            n = SIZES[c]
            rows = pl.ds(r_me + OFFS[c], n)
            vrows = pl.ds(0, n)
            cp_p = pltpu.make_async_copy(p_ref.at[0, rows], p_vmem.at[vrows],
                                         copy_sems.at[0])
            cp_r = pltpu.make_async_copy(resid_ref.at[rows], resid_vmem.at[vrows],
                                         copy_sems.at[2])
            cp_p.start(); cp_r.start()
            rdma(p_ref.at[0, rows], out_ref.at[rows],
                 p1_send.at[c], p1_recv.at[c], y_nbr).wait_recv()
            cp_s = pltpu.make_async_copy(out_ref.at[rows], s_vmem.at[vrows],
                                         copy_sems.at[1])
            cp_s.start()
            cp_p.wait(); cp_r.wait(); cp_s.wait()

            yv = p_vmem[vrows, :] + s_vmem[vrows, :] + resid_vmem[vrows, :]
            rms = jnp.sqrt(jnp.mean(yv * yv, axis=-1, keepdims=True) + 1e-6)
            o_vmem[vrows, :] = yv / rms * g_ref[...]

            cp_o = pltpu.make_async_copy(o_vmem.at[vrows], out_ref.at[rows],
                                         copy_sems.at[3])
            cp_o.start(); cp_o.wait()

            rdma(out_ref.at[rows], out_ref.at[rows],
                 a_send.at[c], a_recv.at[c], y_nbr).start()
            rdma(out_ref.at[rows], out_ref.at[rows],
                 b_own_send.at[c], b_own_recv.at[c], x_nbr).start()

        for c in range(NC):
            rows = pl.ds(r_yn + OFFS[c], SIZES[c])
            rdma(out_ref.at[rows], out_ref.at[rows],
                 a_send.at[c], a_recv.at[c], y_nbr).wait_recv()
            n = SIZES[c] if c != LAST else SIZES[c] - REROUTE
            rows_f = pl.ds(r_yn + OFFS[c], n)
            rdma(out_ref.at[rows_f], out_ref.at[rows_f],
                 b_fwd_send.at[c], b_fwd_recv.at[c], x_nbr).start()

        for c in range(NC):
            rows = pl.ds(q_xn * Q_ROWS + OFFS[c], SIZES[c])
            rdma(out_ref.at[rows], out_ref.at[rows],
                 b_own_send.at[c], b_own_recv.at[c], x_nbr).wait_recv()
        rows_c = pl.ds(q_xn * Q_ROWS + Q_ROWS - REROUTE, REROUTE)
        rdma(out_ref.at[rows_c], out_ref.at[rows_c],
             c_send, c_recv, y_nbr).start()

        for c in range(NC):
            n = SIZES[c] if c != LAST else SIZES[c] - REROUTE
            rows = pl.ds(q_dg * Q_ROWS + OFFS[c], n)
            rdma(out_ref.at[rows], out_ref.at[rows],
                 b_fwd_send.at[c], b_fwd_recv.at[c], x_nbr).wait_recv()
        rows_d = pl.ds(q_dg * Q_ROWS + Q_ROWS - REROUTE, REROUTE)
        rdma(out_ref.at[rows_d], out_ref.at[rows_d],
             c_send, c_recv, y_nbr).wait_recv()

        rdma(out_ref.at[rows_c], out_ref.at[rows_c],
             c_send, c_recv, y_nbr).wait_send()
        for c in range(NC):
            rows = pl.ds(r_yn + OFFS[c], SIZES[c])
            rdma(p_ref.at[0, rows], out_ref.at[rows],
                 p1_send.at[c], p1_recv.at[c], y_nbr).wait_send()
            n = SIZES[c] if c != LAST else SIZES[c] - REROUTE
            rows_f = pl.ds(r_yn + OFFS[c], n)
            rdma(out_ref.at[rows_f], out_ref.at[rows_f],
                 b_fwd_send.at[c], b_fwd_recv.at[c], x_nbr).wait_send()
            rows = pl.ds(r_me + OFFS[c], SIZES[c])
            rdma(out_ref.at[rows], out_ref.at[rows],
                 a_send.at[c], a_recv.at[c], y_nbr).wait_send()
            rdma(out_ref.at[rows], out_ref.at[rows],
                 b_own_send.at[c], b_own_recv.at[c], x_nbr).wait_send()

    sem_arr = pltpu.SemaphoreType.DMA((NC,))
    return pl.pallas_call(
        body,
        out_shape=jax.ShapeDtypeStruct((N_ROWS, N_COLS), jnp.float32),
        in_specs=[
            pl.BlockSpec(memory_space=pl.MemorySpace.ANY),
            pl.BlockSpec(memory_space=pl.MemorySpace.ANY),
            pl.BlockSpec(memory_space=pltpu.MemorySpace.VMEM),
        ],
        out_specs=pl.BlockSpec(memory_space=pl.MemorySpace.ANY),
        scratch_shapes=[
            pltpu.VMEM((MAXCH, N_COLS), jnp.float32),
            pltpu.VMEM((MAXCH, N_COLS), jnp.float32),
            pltpu.VMEM((MAXCH, N_COLS), jnp.float32),
            pltpu.VMEM((MAXCH, N_COLS), jnp.float32),
            sem_arr, sem_arr, sem_arr, sem_arr,
            sem_arr, sem_arr, sem_arr, sem_arr,
            pltpu.SemaphoreType.DMA,
            pltpu.SemaphoreType.DMA,
            pltpu.SemaphoreType.DMA((4,)),
        ],
        compiler_params=pltpu.CompilerParams(
            collective_id=0, vmem_limit_bytes=96 * 1024 * 1024),
    )(partial, resid, g)


# device time: 434420 ns/iter; 1.0045x vs baseline; 1.0045x over previous
import jax
import jax.numpy as jnp
from jax import lax
from jax.experimental import pallas as pl
from jax.experimental.pallas import tpu as pltpu

N_ROWS = 4096
N_COLS = 4096
Q_ROWS = N_ROWS // 4
NC = 8
CH = Q_ROWS // NC
LAST = NC - 1


def _identity(x):
    def body(x_ref, o_ref):
        pass

    return pl.pallas_call(
        body,
        out_shape=jax.ShapeDtypeStruct(x.shape, x.dtype),
        in_specs=[pl.BlockSpec(memory_space=pl.MemorySpace.ANY)],
        out_specs=pl.BlockSpec(memory_space=pl.MemorySpace.ANY),
        input_output_aliases={0: 0},
    )(x)


def kernel(partial, resid, gamma):
    g = gamma.reshape(1, N_COLS)

    def body(p_ref, resid_ref, g_ref, out_ref,
             p_vmem, s_vmem, resid_vmem, o_vmem,
             p1_send, p1_recv, a_send, a_recv,
             b_own_send, b_own_recv, b_fwd_send, b_fwd_recv,
             c_send, c_recv, copy_sems):
        my_x = lax.axis_index("x")
        my_y = lax.axis_index("y")
        y_nbr = (my_x, 1 - my_y)
        x_nbr = (1 - my_x, my_y)

        q_me = 2 * my_x + my_y
        q_yn = 2 * my_x + (1 - my_y)
        q_xn = 2 * (1 - my_x) + my_y
        q_dg = 2 * (1 - my_x) + (1 - my_y)
        r_me = q_me * Q_ROWS
        r_yn = q_yn * Q_ROWS

        def rdma(src, dst, ssem, rsem, dev):
            return pltpu.make_async_remote_copy(
                src_ref=src, dst_ref=dst, send_sem=ssem, recv_sem=rsem,
                device_id=dev, device_id_type=pl.DeviceIdType.MESH)

        barrier = pltpu.get_barrier_semaphore()
        for nbr in (y_nbr, x_nbr):
            pl.semaphore_signal(barrier, inc=1, device_id=nbr,
                                device_id_type=pl.DeviceIdType.MESH)
        pl.semaphore_wait(barrier, 2)

        for c in range(NC):
            rows = pl.ds(r_yn + c * CH, CH)
            rdma(p_ref.at[0, rows], out_ref.at[rows],
                 p1_send.at[c], p1_recv.at[c], y_nbr).start()

        for c in range(NC):
            rows = pl.ds(r_me + c * CH, CH)
            cp_p = pltpu.make_async_copy(p_ref.at[0, rows], p_vmem, copy_sems.at[0])
            cp_r = pltpu.make_async_copy(resid_ref.at[rows], resid_vmem, copy_sems.at[2])
            cp_p.start(); cp_r.start()
            rdma(p_ref.at[0, rows], out_ref.at[rows],
                 p1_send.at[c], p1_recv.at[c], y_nbr).wait_recv()
            cp_s = pltpu.make_async_copy(out_ref.at[rows], s_vmem, copy_sems.at[1])
            cp_s.start()
            cp_p.wait(); cp_r.wait(); cp_s.wait()

            yv = p_vmem[...] + s_vmem[...] + resid_vmem[...]
            rms = jnp.sqrt(jnp.mean(yv * yv, axis=-1, keepdims=True) + 1e-6)
            o_vmem[...] = yv / rms * g_ref[...]

            cp_o = pltpu.make_async_copy(o_vmem, out_ref.at[rows], copy_sems.at[3])
            cp_o.start(); cp_o.wait()

            rdma(out_ref.at[rows], out_ref.at[rows],
                 a_send.at[c], a_recv.at[c], y_nbr).start()
            rdma(out_ref.at[rows], out_ref.at[rows],
                 b_own_send.at[c], b_own_recv.at[c], x_nbr).start()

        for c in range(NC):
            rows = pl.ds(r_yn + c * CH, CH)
            rdma(out_ref.at[rows], out_ref.at[rows],
                 a_send.at[c], a_recv.at[c], y_nbr).wait_recv()
            if c != LAST:
                rdma(out_ref.at[rows], out_ref.at[rows],
                     b_fwd_send.at[c], b_fwd_recv.at[c], x_nbr).start()

        for c in range(NC):
            rows = pl.ds(q_xn * Q_ROWS + c * CH, CH)
            rdma(out_ref.at[rows], out_ref.at[rows],
                 b_own_send.at[c], b_own_recv.at[c], x_nbr).wait_recv()
        rows_c = pl.ds(q_xn * Q_ROWS + LAST * CH, CH)
        rdma(out_ref.at[rows_c], out_ref.at[rows_c],
             c_send, c_recv, y_nbr).start()

        for c in range(NC - 1):
            rows = pl.ds(q_dg * Q_ROWS + c * CH, CH)
            rdma(out_ref.at[rows], out_ref.at[rows],
                 b_fwd_send.at[c], b_fwd_recv.at[c], x_nbr).wait_recv()
        rows_d = pl.ds(q_dg * Q_ROWS + LAST * CH, CH)
        rdma(out_ref.at[rows_d], out_ref.at[rows_d],
             c_send, c_recv, y_nbr).wait_recv()

        rdma(out_ref.at[rows_c], out_ref.at[rows_c],
             c_send, c_recv, y_nbr).wait_send()
        for c in range(NC):
            rows = pl.ds(r_yn + c * CH, CH)
            rdma(p_ref.at[0, rows], out_ref.at[rows],
                 p1_send.at[c], p1_recv.at[c], y_nbr).wait_send()
            if c != LAST:
                rdma(out_ref.at[rows], out_ref.at[rows],
                     b_fwd_send.at[c], b_fwd_recv.at[c], x_nbr).wait_send()
            rows = pl.ds(r_me + c * CH, CH)
            rdma(out_ref.at[rows], out_ref.at[rows],
                 a_send.at[c], a_recv.at[c], y_nbr).wait_send()
            rdma(out_ref.at[rows], out_ref.at[rows],
                 b_own_send.at[c], b_own_recv.at[c], x_nbr).wait_send()

    sem_arr = pltpu.SemaphoreType.DMA((NC,))
    return pl.pallas_call(
        body,
        out_shape=jax.ShapeDtypeStruct((N_ROWS, N_COLS), jnp.float32),
        in_specs=[
            pl.BlockSpec(memory_space=pl.MemorySpace.ANY),
            pl.BlockSpec(memory_space=pl.MemorySpace.ANY),
            pl.BlockSpec(memory_space=pltpu.MemorySpace.VMEM),
        ],
        out_specs=pl.BlockSpec(memory_space=pl.MemorySpace.ANY),
        scratch_shapes=[
            pltpu.VMEM((CH, N_COLS), jnp.float32),
            pltpu.VMEM((CH, N_COLS), jnp.float32),
            pltpu.VMEM((CH, N_COLS), jnp.float32),
            pltpu.VMEM((CH, N_COLS), jnp.float32),
            sem_arr, sem_arr, sem_arr, sem_arr,
            sem_arr, sem_arr, sem_arr, sem_arr,
            pltpu.SemaphoreType.DMA,
            pltpu.SemaphoreType.DMA,
            pltpu.SemaphoreType.DMA((4,)),
        ],
        compiler_params=pltpu.CompilerParams(
            collective_id=0, vmem_limit_bytes=96 * 1024 * 1024),
    )(partial, resid, g)


def kernel(partial, resid, gamma, _comm=kernel):
    return _identity(_comm(partial, resid, gamma))


# device time: 423657 ns/iter; 1.0300x vs baseline; 1.0254x over previous
import jax
import jax.numpy as jnp
from jax import lax
from jax.experimental import pallas as pl
from jax.experimental.pallas import tpu as pltpu

N_ROWS = 4096
N_COLS = 4096
Q_ROWS = N_ROWS // 4
NC = 16
CH = Q_ROWS // NC
LAST = NC - 1


def kernel(partial, resid, gamma):
    g = gamma.reshape(1, N_COLS)

    def body(p_ref, resid_ref, g_ref, out_ref,
             p_vmem, s_vmem, resid_vmem, o_vmem,
             p1_send, p1_recv, a_send, a_recv,
             b_own_send, b_own_recv, b_fwd_send, b_fwd_recv,
             c_send, c_recv, copy_sems):
        my_x = lax.axis_index("x")
        my_y = lax.axis_index("y")
        y_nbr = (my_x, 1 - my_y)
        x_nbr = (1 - my_x, my_y)

        q_me = 2 * my_x + my_y
        q_yn = 2 * my_x + (1 - my_y)
        q_xn = 2 * (1 - my_x) + my_y
        q_dg = 2 * (1 - my_x) + (1 - my_y)
        r_me = q_me * Q_ROWS
        r_yn = q_yn * Q_ROWS

        def rdma(src, dst, ssem, rsem, dev):
            return pltpu.make_async_remote_copy(
                src_ref=src, dst_ref=dst, send_sem=ssem, recv_sem=rsem,
                device_id=dev, device_id_type=pl.DeviceIdType.MESH)

        barrier = pltpu.get_barrier_semaphore()
        for nbr in (y_nbr, x_nbr):
            pl.semaphore_signal(barrier, inc=1, device_id=nbr,
                                device_id_type=pl.DeviceIdType.MESH)
        pl.semaphore_wait(barrier, 2)

        for c in range(NC):
            rows = pl.ds(r_yn + c * CH, CH)
            rdma(p_ref.at[0, rows], out_ref.at[rows],
                 p1_send.at[c], p1_recv.at[c], y_nbr).start()

        for c in range(NC):
            rows = pl.ds(r_me + c * CH, CH)
            cp_p = pltpu.make_async_copy(p_ref.at[0, rows], p_vmem, copy_sems.at[0])
            cp_r = pltpu.make_async_copy(resid_ref.at[rows], resid_vmem, copy_sems.at[2])
            cp_p.start(); cp_r.start()
            rdma(p_ref.at[0, rows], out_ref.at[rows],
                 p1_send.at[c], p1_recv.at[c], y_nbr).wait_recv()
            cp_s = pltpu.make_async_copy(out_ref.at[rows], s_vmem, copy_sems.at[1])
            cp_s.start()
            cp_p.wait(); cp_r.wait(); cp_s.wait()

            yv = p_vmem[...] + s_vmem[...] + resid_vmem[...]
            rms = jnp.sqrt(jnp.mean(yv * yv, axis=-1, keepdims=True) + 1e-6)
            o_vmem[...] = yv / rms * g_ref[...]

            cp_o = pltpu.make_async_copy(o_vmem, out_ref.at[rows], copy_sems.at[3])
            cp_o.start(); cp_o.wait()

            rdma(out_ref.at[rows], out_ref.at[rows],
                 a_send.at[c], a_recv.at[c], y_nbr).start()
            rdma(out_ref.at[rows], out_ref.at[rows],
                 b_own_send.at[c], b_own_recv.at[c], x_nbr).start()

        for c in range(NC):
            rows = pl.ds(r_yn + c * CH, CH)
            rdma(out_ref.at[rows], out_ref.at[rows],
                 a_send.at[c], a_recv.at[c], y_nbr).wait_recv()
            if c != LAST:
                rdma(out_ref.at[rows], out_ref.at[rows],
                     b_fwd_send.at[c], b_fwd_recv.at[c], x_nbr).start()

        for c in range(NC):
            rows = pl.ds(q_xn * Q_ROWS + c * CH, CH)
            rdma(out_ref.at[rows], out_ref.at[rows],
                 b_own_send.at[c], b_own_recv.at[c], x_nbr).wait_recv()
        rows_c = pl.ds(q_xn * Q_ROWS + LAST * CH, CH)
        rdma(out_ref.at[rows_c], out_ref.at[rows_c],
             c_send, c_recv, y_nbr).start()

        for c in range(NC - 1):
            rows = pl.ds(q_dg * Q_ROWS + c * CH, CH)
            rdma(out_ref.at[rows], out_ref.at[rows],
                 b_fwd_send.at[c], b_fwd_recv.at[c], x_nbr).wait_recv()
        rows_d = pl.ds(q_dg * Q_ROWS + LAST * CH, CH)
        rdma(out_ref.at[rows_d], out_ref.at[rows_d],
             c_send, c_recv, y_nbr).wait_recv()

        rdma(out_ref.at[rows_c], out_ref.at[rows_c],
             c_send, c_recv, y_nbr).wait_send()
        for c in range(NC):
            rows = pl.ds(r_yn + c * CH, CH)
            rdma(p_ref.at[0, rows], out_ref.at[rows],
                 p1_send.at[c], p1_recv.at[c], y_nbr).wait_send()
            if c != LAST:
                rdma(out_ref.at[rows], out_ref.at[rows],
                     b_fwd_send.at[c], b_fwd_recv.at[c], x_nbr).wait_send()
            rows = pl.ds(r_me + c * CH, CH)
            rdma(out_ref.at[rows], out_ref.at[rows],
                 a_send.at[c], a_recv.at[c], y_nbr).wait_send()
            rdma(out_ref.at[rows], out_ref.at[rows],
                 b_own_send.at[c], b_own_recv.at[c], x_nbr).wait_send()

    sem_arr = pltpu.SemaphoreType.DMA((NC,))
    return pl.pallas_call(
        body,
        out_shape=jax.ShapeDtypeStruct((N_ROWS, N_COLS), jnp.float32),
        in_specs=[
            pl.BlockSpec(memory_space=pl.MemorySpace.ANY),
            pl.BlockSpec(memory_space=pl.MemorySpace.ANY),
            pl.BlockSpec(memory_space=pltpu.MemorySpace.VMEM),
        ],
        out_specs=pl.BlockSpec(memory_space=pl.MemorySpace.ANY),
        scratch_shapes=[
            pltpu.VMEM((CH, N_COLS), jnp.float32),
            pltpu.VMEM((CH, N_COLS), jnp.float32),
            pltpu.VMEM((CH, N_COLS), jnp.float32),
            pltpu.VMEM((CH, N_COLS), jnp.float32),
            sem_arr, sem_arr, sem_arr, sem_arr,
            sem_arr, sem_arr, sem_arr, sem_arr,
            pltpu.SemaphoreType.DMA,
            pltpu.SemaphoreType.DMA,
            pltpu.SemaphoreType.DMA((4,)),
        ],
        compiler_params=pltpu.CompilerParams(
            collective_id=0, vmem_limit_bytes=96 * 1024 * 1024),
    )(partial, resid, g)


# device time: 419262 ns/iter; 1.0408x vs baseline; 1.0105x over previous
import jax
import jax.numpy as jnp
from jax import lax
from jax.experimental import pallas as pl
from jax.experimental.pallas import tpu as pltpu

N_ROWS = 4096
N_COLS = 4096
Q_ROWS = N_ROWS // 4
NC = 32
CH = Q_ROWS // NC
LAST = NC - 1


def kernel(partial, resid, gamma):
    g = gamma.reshape(1, N_COLS)

    def body(p_ref, resid_ref, g_ref, out_ref,
             p_vmem, s_vmem, resid_vmem, o_vmem,
             p1_send, p1_recv, a_send, a_recv,
             b_own_send, b_own_recv, b_fwd_send, b_fwd_recv,
             c_send, c_recv, copy_sems):
        my_x = lax.axis_index("x")
        my_y = lax.axis_index("y")
        y_nbr = (my_x, 1 - my_y)
        x_nbr = (1 - my_x, my_y)

        q_me = 2 * my_x + my_y
        q_yn = 2 * my_x + (1 - my_y)
        q_xn = 2 * (1 - my_x) + my_y
        q_dg = 2 * (1 - my_x) + (1 - my_y)
        r_me = q_me * Q_ROWS
        r_yn = q_yn * Q_ROWS

        def rdma(src, dst, ssem, rsem, dev):
            return pltpu.make_async_remote_copy(
                src_ref=src, dst_ref=dst, send_sem=ssem, recv_sem=rsem,
                device_id=dev, device_id_type=pl.DeviceIdType.MESH)

        barrier = pltpu.get_barrier_semaphore()
        for nbr in (y_nbr, x_nbr):
            pl.semaphore_signal(barrier, inc=1, device_id=nbr,
                                device_id_type=pl.DeviceIdType.MESH)
        pl.semaphore_wait(barrier, 2)

        for c in range(NC):
            rows = pl.ds(r_yn + c * CH, CH)
            rdma(p_ref.at[0, rows], out_ref.at[rows],
                 p1_send.at[c], p1_recv.at[c], y_nbr).start()

        for c in range(NC):
            rows = pl.ds(r_me + c * CH, CH)
            cp_p = pltpu.make_async_copy(p_ref.at[0, rows], p_vmem, copy_sems.at[0])
            cp_r = pltpu.make_async_copy(resid_ref.at[rows], resid_vmem, copy_sems.at[2])
            cp_p.start(); cp_r.start()
            rdma(p_ref.at[0, rows], out_ref.at[rows],
                 p1_send.at[c], p1_recv.at[c], y_nbr).wait_recv()
            cp_s = pltpu.make_async_copy(out_ref.at[rows], s_vmem, copy_sems.at[1])
            cp_s.start()
            cp_p.wait(); cp_r.wait(); cp_s.wait()

            yv = p_vmem[...] + s_vmem[...] + resid_vmem[...]
            rms = jnp.sqrt(jnp.mean(yv * yv, axis=-1, keepdims=True) + 1e-6)
            o_vmem[...] = yv / rms * g_ref[...]

            cp_o = pltpu.make_async_copy(o_vmem, out_ref.at[rows], copy_sems.at[3])
            cp_o.start(); cp_o.wait()

            rdma(out_ref.at[rows], out_ref.at[rows],
                 a_send.at[c], a_recv.at[c], y_nbr).start()
            rdma(out_ref.at[rows], out_ref.at[rows],
                 b_own_send.at[c], b_own_recv.at[c], x_nbr).start()

        for c in range(NC):
            rows = pl.ds(r_yn + c * CH, CH)
            rdma(out_ref.at[rows], out_ref.at[rows],
                 a_send.at[c], a_recv.at[c], y_nbr).wait_recv()
            if c != LAST:
                rdma(out_ref.at[rows], out_ref.at[rows],
                     b_fwd_send.at[c], b_fwd_recv.at[c], x_nbr).start()

        for c in range(NC):
            rows = pl.ds(q_xn * Q_ROWS + c * CH, CH)
            rdma(out_ref.at[rows], out_ref.at[rows],
                 b_own_send.at[c], b_own_recv.at[c], x_nbr).wait_recv()
        rows_c = pl.ds(q_xn * Q_ROWS + LAST * CH, CH)
        rdma(out_ref.at[rows_c], out_ref.at[rows_c],
             c_send, c_recv, y_nbr).start()

        for c in range(NC - 1):
            rows = pl.ds(q_dg * Q_ROWS + c * CH, CH)
            rdma(out_ref.at[rows], out_ref.at[rows],
                 b_fwd_send.at[c], b_fwd_recv.at[c], x_nbr).wait_recv()
        rows_d = pl.ds(q_dg * Q_ROWS + LAST * CH, CH)
        rdma(out_ref.at[rows_d], out_ref.at[rows_d],
             c_send, c_recv, y_nbr).wait_recv()

        rdma(out_ref.at[rows_c], out_ref.at[rows_c],
             c_send, c_recv, y_nbr).wait_send()
        for c in range(NC):
            rows = pl.ds(r_yn + c * CH, CH)
            rdma(p_ref.at[0, rows], out_ref.at[rows],
                 p1_send.at[c], p1_recv.at[c], y_nbr).wait_send()
            if c != LAST:
                rdma(out_ref.at[rows], out_ref.at[rows],
                     b_fwd_send.at[c], b_fwd_recv.at[c], x_nbr).wait_send()
            rows = pl.ds(r_me + c * CH, CH)
            rdma(out_ref.at[rows], out_ref.at[rows],
                 a_send.at[c], a_recv.at[c], y_nbr).wait_send()
            rdma(out_ref.at[rows], out_ref.at[rows],
                 b_own_send.at[c], b_own_recv.at[c], x_nbr).wait_send()

    sem_arr = pltpu.SemaphoreType.DMA((NC,))
    return pl.pallas_call(
        body,
        out_shape=jax.ShapeDtypeStruct((N_ROWS, N_COLS), jnp.float32),
        in_specs=[
            pl.BlockSpec(memory_space=pl.MemorySpace.ANY),
            pl.BlockSpec(memory_space=pl.MemorySpace.ANY),
            pl.BlockSpec(memory_space=pltpu.MemorySpace.VMEM),
        ],
        out_specs=pl.BlockSpec(memory_space=pl.MemorySpace.ANY),
        scratch_shapes=[
            pltpu.VMEM((CH, N_COLS), jnp.float32),
            pltpu.VMEM((CH, N_COLS), jnp.float32),
            pltpu.VMEM((CH, N_COLS), jnp.float32),
            pltpu.VMEM((CH, N_COLS), jnp.float32),
            sem_arr, sem_arr, sem_arr, sem_arr,
            sem_arr, sem_arr, sem_arr, sem_arr,
            pltpu.SemaphoreType.DMA,
            pltpu.SemaphoreType.DMA,
            pltpu.SemaphoreType.DMA((4,)),
        ],
        compiler_params=pltpu.CompilerParams(
            collective_id=0, vmem_limit_bytes=96 * 1024 * 1024),
    )(partial, resid, g)
